# Initial kernel scaffold; baseline (speedup 1.0000x reference)
import jax
import jax.numpy as jnp
from jax.experimental import pallas as pl


def kernel(token_ids, emb_table):
    raise NotImplementedError("write your pallas kernel here")


# speedup vs baseline: 4.2470x; 4.2470x over previous
import functools

import jax
import jax.numpy as jnp
from jax import lax
from jax.experimental import pallas as pl
from jax.experimental.pallas import tpu as pltpu
from jax.experimental.pallas import tpu_sc as plsc

BATCH = 1024
MAX_LEN = 200
D_MODEL = 128
VOCAB = 100000

NUM_CORES = 2
NUM_SUBCORES = 16
NW = NUM_CORES * NUM_SUBCORES
ROWS_PER_W = (BATCH * MAX_LEN) // NW
SENT_PER_W = ROWS_PER_W // MAX_LEN


def _pe_body(pe_ref):
    pos = lax.broadcasted_iota(jnp.int32, (MAX_LEN, D_MODEL), 0).astype(
        jnp.float32)
    d = lax.broadcasted_iota(jnp.int32, (MAX_LEN, D_MODEL), 1)
    even_i = ((d // 2) * 2).astype(jnp.float32)
    inv_denom = jnp.reciprocal(jnp.power(10000.0, even_i / D_MODEL))
    angle = pos * inv_denom
    pe_ref[...] = jnp.where(d % 2 == 0, jnp.sin(angle), jnp.cos(angle))


def _compute_pe():
    return pl.pallas_call(
        _pe_body,
        out_shape=jax.ShapeDtypeStruct((MAX_LEN, D_MODEL), jnp.float32),
    )()


def _sc_body(tok_hbm, table_hbm, pe_hbm, out_hbm, idx_v, pe_v, rows, sem):
    wid = lax.axis_index("s") * NUM_CORES + lax.axis_index("c")
    base = wid * ROWS_PER_W
    pltpu.sync_copy(tok_hbm.at[pl.ds(base, ROWS_PER_W)], idx_v)
    pltpu.sync_copy(pe_hbm, pe_v)

    @pl.loop(0, SENT_PER_W)
    def _sent(s):
        c0 = pltpu.async_copy(
            table_hbm.at[idx_v.at[pl.ds(s * MAX_LEN, 128)]],
            rows.at[pl.ds(0, 128)], sem)
        c1 = pltpu.async_copy(
            table_hbm.at[idx_v.at[pl.ds(s * MAX_LEN + 128, 72)]],
            rows.at[pl.ds(128, 72)], sem)
        c0.wait()
        c1.wait()

        @pl.loop(0, MAX_LEN)
        def _row(r):
            for c in range(D_MODEL // 16):
                plsc.addupdate(rows.at[r, pl.ds(c * 16, 16)],
                               pe_v[r, pl.ds(c * 16, 16)])

        pltpu.sync_copy(rows,
                        out_hbm.at[pl.ds(base + s * MAX_LEN, MAX_LEN)])


@functools.partial(
    pl.kernel,
    out_type=jax.ShapeDtypeStruct((BATCH * MAX_LEN, D_MODEL), jnp.float32),
    mesh=plsc.VectorSubcoreMesh(core_axis_name="c", subcore_axis_name="s",
                                num_cores=NUM_CORES,
                                num_subcores=NUM_SUBCORES),
    scratch_types=[
        pltpu.VMEM((ROWS_PER_W,), jnp.int32),
        pltpu.VMEM((MAX_LEN, D_MODEL), jnp.float32),
        pltpu.VMEM((MAX_LEN, D_MODEL), jnp.float32),
        pltpu.SemaphoreType.DMA,
    ],
)
def _sc_embed(tok_hbm, table_hbm, pe_hbm, out_hbm, idx_v, pe_v, rows, sem):
    _sc_body(tok_hbm, table_hbm, pe_hbm, out_hbm, idx_v, pe_v, rows, sem)


@jax.jit
def kernel(token_ids, emb_table):
    pe = _compute_pe()
    flat = token_ids.reshape(BATCH * MAX_LEN)
    out = _sc_embed(flat, emb_table, pe)
    return out.reshape(BATCH, MAX_LEN, D_MODEL)
